# baseline (speedup 1.0000x reference)
import functools

import jax
import jax.numpy as jnp
from jax.experimental import pallas as pl
from jax.experimental.pallas import tpu as pltpu

NUM_ADDRESSES = 100000
ADDRESS_DIM = 512
CONTENT_DIM = 512
BATCH = 256
THRESHOLD = 76
BLK = 2000


def _fused_body(address_ref, addresses_ref, content_ref, out_ref, acc_ref):
    j = pl.program_id(0)

    @pl.when(j == 0)
    def _init():
        acc_ref[...] = jnp.zeros_like(acc_ref)

    sim = jax.lax.dot_general(
        address_ref[...], addresses_ref[...],
        (((1,), (1,)), ((), ())),
        preferred_element_type=jnp.float32,
    )
    mask = (sim >= THRESHOLD).astype(jnp.float32)
    acc_ref[...] += jax.lax.dot_general(
        mask, content_ref[...],
        (((1,), (0,)), ((), ())),
        preferred_element_type=jnp.float32,
    )

    @pl.when(j == pl.num_programs(0) - 1)
    def _emit():
        out_ref[...] = jnp.sign(acc_ref[...])


@jax.jit
def kernel(address, addresses, content):
    grid = (NUM_ADDRESSES // BLK,)
    return pl.pallas_call(
        _fused_body,
        grid=grid,
        in_specs=[
            pl.BlockSpec((BATCH, ADDRESS_DIM), lambda j: (0, 0)),
            pl.BlockSpec((BLK, ADDRESS_DIM), lambda j: (j, 0)),
            pl.BlockSpec((BLK, CONTENT_DIM), lambda j: (j, 0)),
        ],
        out_specs=pl.BlockSpec((BATCH, CONTENT_DIM), lambda j: (0, 0)),
        out_shape=jax.ShapeDtypeStruct((BATCH, CONTENT_DIM), jnp.float32),
        scratch_shapes=[pltpu.VMEM((BATCH, CONTENT_DIM), jnp.float32)],
    )(address, addresses, content)
